# Initial kernel scaffold; baseline (speedup 1.0000x reference)
import jax
import jax.numpy as jnp
from jax.experimental import pallas as pl


def kernel(positions, senders, receivers, species, Wemb, W1_0, Wself_0, W2_0, W1_1, Wself_1, W2_1, W1_2, Wself_2, W2_2, Wout1, Wout2):
    raise NotImplementedError("write your pallas kernel here")


# speedup vs baseline: 1.5759x; 1.5759x over previous
import math

import jax
import jax.numpy as jnp
from jax.experimental import pallas as pl

N = 64
E = 896
F = 128
NB = 8
CUTOFF = 5.0
AVG_NEIGH = 14.26
B = N * 3
BC = 8
_INV_AVG = 1.0 / AVG_NEIGH
_EPS = 1e-6


def _dot(a, b):
    bf = jnp.bfloat16
    f32 = jnp.float32
    ah = a.astype(bf)
    al = (a - ah.astype(f32)).astype(bf)
    bh = b.astype(bf)
    bl = (b - bh.astype(f32)).astype(bf)

    def d(x, y):
        return jnp.dot(x, y, preferred_element_type=f32)

    return d(ah, bh) + (d(ah, bl) + d(al, bh))


def _silu_d(x):
    sg = jax.nn.sigmoid(x)
    sp = sg * (1.0 + x * (1.0 - sg))
    spp = sg * (1.0 - sg) * (2.0 + x * (1.0 - 2.0 * sg))
    return sp, spp


def _primal_kernel(pos, ohsp, wemb, dmat, smat, stmat, rmat, rtmat,
                   w1h, w1rb, w1v, w2, ws, w1ht, w1rbt, w2t, wst,
                   wout1t, wout2t,
                   spu_o, bu_o, spz_o, bz_o, fp_o, vhat_o, cor_o, d2_o):
    v = _dot(dmat[...], pos[...])
    r2 = jnp.sum(v * v, axis=1, keepdims=True) + _EPS
    r = jnp.sqrt(r2)
    inv_r = 1.0 / r
    kk = ((jax.lax.broadcasted_iota(jnp.int32, (1, NB), 1) + 1)
          .astype(jnp.float32) * math.pi)
    kr = r * kk
    sk = jnp.sin(kr)
    ck = jnp.cos(kr)
    s0 = sk * inv_r
    s1 = kk * ck * inv_r - sk * (inv_r * inv_r)
    s2 = (-(kk * kk) * sk * inv_r - 2.0 * kk * ck * (inv_r * inv_r)
          + 2.0 * sk * (inv_r * inv_r * inv_r))
    mask = r < 1.0
    env0 = jnp.where(mask, (1.0 - r) ** 2, 0.0)
    env1 = jnp.where(mask, -2.0 * (1.0 - r), 0.0)
    env2 = jnp.where(mask, 2.0, 0.0)
    rb = s0 * env0
    fp = s1 * env0 + s0 * env1
    fpp = s2 * env0 + 2.0 * s1 * env1 + s0 * env2

    h = _dot(ohsp[...], wemb[...])
    us = []
    zs = []
    for l in range(3):
        hs = _dot(smat[...], h)
        u = _dot(hs, w1h[l]) + _dot(rb, w1rb[l]) + _dot(v, w1v[l])
        us.append(u)
        a = u * jax.nn.sigmoid(u)
        agg = _dot(rtmat[...], a) * _INV_AVG
        z = _dot(agg, w2[l]) + _dot(h, ws[l])
        zs.append(z)
        h = z * jax.nn.sigmoid(z)

    ones_n = jnp.ones((N, 1), dtype=jnp.float32)
    gh = _dot(_dot(ones_n, wout2t[...]), wout1t[...])
    g_rb = jnp.zeros((E, NB), dtype=jnp.float32)
    for l in (2, 1, 0):
        spz, spz2 = _silu_d(zs[l])
        spz_o[l] = spz
        bz_o[l] = gh * spz2
        gz = gh * spz
        g_agg = _dot(gz, w2t[l])
        gh = _dot(gz, wst[l])
        g_a = _dot(rmat[...], g_agg) * _INV_AVG
        spu, spu2 = _silu_d(us[l])
        spu_o[l] = spu
        bu_o[l] = g_a * spu2
        gu = g_a * spu
        gh = gh + _dot(stmat[...], _dot(gu, w1ht[l]))
        g_rb = g_rb + _dot(gu, w1rbt[l])

    c = jnp.sum(g_rb * fp, axis=1, keepdims=True)
    fp_o[...] = fp
    vhat_o[...] = v * inv_r
    cor_o[...] = c * inv_r
    d2_o[...] = jnp.sum(g_rb * fpp, axis=1, keepdims=True)


def _tangent_kernel(smat, stmat, rmat, rtmat, dtmat,
                    w1h, w1rb, w1v, w2, ws, w1ht, w1rbt, w1vt, w2t, wst,
                    spu, bu, spz, bz, fp_r, vhat_r, cor_r, d2_r, tv_r,
                    out):
    tv = tv_r[...]
    vhat = vhat_r[...]
    fp = fp_r[...]
    tr = jnp.sum(tv * vhat[:, None, :], axis=2)
    trb2 = (tr[:, :, None] * fp[:, None, :]).reshape(E * BC, NB)
    tv2 = tv.reshape(E * BC, 3)

    th = jnp.zeros((N, BC, F), dtype=jnp.float32)
    tus = []
    tzs = []
    for l in range(3):
        tu2 = _dot(trb2, w1rb[l]) + _dot(tv2, w1v[l])
        if l > 0:
            ths = _dot(smat[...], th.reshape(N, BC * F)).reshape(E * BC, F)
            tu2 = tu2 + _dot(ths, w1h[l])
        tu = tu2.reshape(E, BC, F)
        tus.append(tu)
        ta = spu[l][:, None, :] * tu
        tagg = (_dot(rtmat[...], ta.reshape(E, BC * F)) * _INV_AVG)
        tz2 = _dot(tagg.reshape(N * BC, F), w2[l])
        if l > 0:
            tz2 = tz2 + _dot(th.reshape(N * BC, F), ws[l])
        tz = tz2.reshape(N, BC, F)
        tzs.append(tz)
        th = spz[l][:, None, :] * tz

    tgh = jnp.zeros((N, BC, F), dtype=jnp.float32)
    tgrb = jnp.zeros((E, BC, NB), dtype=jnp.float32)
    tgvd = jnp.zeros((E, BC, 3), dtype=jnp.float32)
    for l in (2, 1, 0):
        tgz = tgh * spz[l][:, None, :] + bz[l][:, None, :] * tzs[l]
        tgz2 = tgz.reshape(N * BC, F)
        tgagg = _dot(tgz2, w2t[l]).reshape(N, BC * F)
        tga = (_dot(rmat[...], tgagg) * _INV_AVG).reshape(E, BC, F)
        tgu = tga * spu[l][:, None, :] + bu[l][:, None, :] * tus[l]
        tgu2 = tgu.reshape(E * BC, F)
        if l > 0:
            tghs = _dot(tgu2, w1ht[l]).reshape(E, BC * F)
            tgh = (_dot(tgz2, wst[l]).reshape(N, BC, F)
                   + _dot(stmat[...], tghs).reshape(N, BC, F))
        tgrb = tgrb + _dot(tgu2, w1rbt[l]).reshape(E, BC, NB)
        tgvd = tgvd + _dot(tgu2, w1vt[l]).reshape(E, BC, 3)

    d2 = d2_r[...]
    cor = cor_r[...]
    tc = jnp.sum(tgrb * fp[:, None, :], axis=2) + d2 * tr
    tgv = (tgvd + tc[:, :, None] * vhat[:, None, :]
           + cor[:, :, None] * (tv - vhat[:, None, :] * tr[:, :, None]))
    out[...] = _dot(dtmat[...], tgv.reshape(E, BC * 3)).reshape(N, BC, 3)


def kernel(positions, senders, receivers, species, Wemb,
           W1_0, Wself_0, W2_0, W1_1, Wself_1, W2_1, W1_2, Wself_2, W2_2,
           Wout1, Wout2):
    f32 = jnp.float32
    S = jax.nn.one_hot(senders, N, dtype=f32)
    R = jax.nn.one_hot(receivers, N, dtype=f32)
    D = (R - S) * (1.0 / CUTOFF)
    TV = (D[:, :, None, None] * jnp.eye(3, dtype=f32)).reshape(E, B, 3)
    OH = jax.nn.one_hot(species, 9, dtype=f32)

    W1s = (W1_0, W1_1, W1_2)
    w1h = jnp.stack([w[:F] for w in W1s])
    w1rb = jnp.stack([w[F:F + NB] for w in W1s])
    w1v = jnp.stack([w[F + NB:] for w in W1s])
    w2 = jnp.stack((W2_0, W2_1, W2_2))
    ws = jnp.stack((Wself_0, Wself_1, Wself_2))
    w1ht = w1h.transpose(0, 2, 1)
    w1rbt = w1rb.transpose(0, 2, 1)
    w1vt = w1v.transpose(0, 2, 1)
    w2t = w2.transpose(0, 2, 1)
    wst = ws.transpose(0, 2, 1)

    spu, bu, spz, bz, fp, vhat, cor, d2 = pl.pallas_call(
        _primal_kernel,
        out_shape=(
            jax.ShapeDtypeStruct((3, E, F), f32),
            jax.ShapeDtypeStruct((3, E, F), f32),
            jax.ShapeDtypeStruct((3, N, F), f32),
            jax.ShapeDtypeStruct((3, N, F), f32),
            jax.ShapeDtypeStruct((E, NB), f32),
            jax.ShapeDtypeStruct((E, 3), f32),
            jax.ShapeDtypeStruct((E, 1), f32),
            jax.ShapeDtypeStruct((E, 1), f32),
        ),
    )(positions, OH, Wemb, D, S, S.T, R, R.T,
      w1h, w1rb, w1v, w2, ws, w1ht, w1rbt, w2t, wst,
      Wout1.T, Wout2.T)

    full = lambda shp: pl.BlockSpec(shp, lambda g: (0,) * len(shp))
    ins = (S, S.T, R, R.T, D.T,
           w1h, w1rb, w1v, w2, ws, w1ht, w1rbt, w1vt, w2t, wst,
           spu, bu, spz, bz, fp, vhat, cor, d2)
    hout = pl.pallas_call(
        _tangent_kernel,
        grid=(B // BC,),
        in_specs=[full(x.shape) for x in ins]
        + [pl.BlockSpec((E, BC, 3), lambda g: (0, g, 0))],
        out_specs=pl.BlockSpec((N, BC, 3), lambda g: (0, g, 0)),
        out_shape=jax.ShapeDtypeStruct((N, B, 3), f32),
    )(*ins, TV)

    return hout.transpose(1, 0, 2).reshape(N, 3, N, 3)


# speedup vs baseline: 2.0026x; 1.2707x over previous
import math

import jax
import jax.numpy as jnp
from jax.experimental import pallas as pl

N = 64
E = 896
F = 128
NB = 8
CUTOFF = 5.0
AVG_NEIGH = 14.26
B = N * 3
BC = 8
_INV_AVG = 1.0 / AVG_NEIGH
_EPS = 1e-6


def _dot(a, b):
    bf = jnp.bfloat16
    f32 = jnp.float32
    ah = a.astype(bf)
    al = (a - ah.astype(f32)).astype(bf)
    bh = b.astype(bf)
    bl = (b - bh.astype(f32)).astype(bf)

    def d(x, y):
        return jnp.dot(x, y, preferred_element_type=f32)

    return d(ah, bh) + (d(ah, bl) + d(al, bh))


def _dot_ex(a, b):
    bf = jnp.bfloat16
    f32 = jnp.float32
    ab = a.astype(bf)
    bh = b.astype(bf)
    bl = (b - bh.astype(f32)).astype(bf)

    def d(x, y):
        return jnp.dot(x, y, preferred_element_type=f32)

    return d(ab, bh) + d(ab, bl)


def _silu_d(x):
    sg = jax.nn.sigmoid(x)
    sp = sg * (1.0 + x * (1.0 - sg))
    spp = sg * (1.0 - sg) * (2.0 + x * (1.0 - 2.0 * sg))
    return sp, spp


def _primal_kernel(pos, ohsp, wemb, rsmat, smat, stmat, rmat, rtmat,
                   w1h, w1rb, w1v, w2, ws, w1ht, w1rbt, w2t, wst,
                   wout1t, wout2t,
                   spu_o, bu_o, spz_o, bz_o, fp_o, vhat_o, cor_o, d2_o):
    v = _dot_ex(rsmat[...], pos[...]) * (1.0 / CUTOFF)
    r2 = jnp.sum(v * v, axis=1, keepdims=True) + _EPS
    r = jnp.sqrt(r2)
    inv_r = 1.0 / r
    kk = ((jax.lax.broadcasted_iota(jnp.int32, (1, NB), 1) + 1)
          .astype(jnp.float32) * math.pi)
    kr = r * kk
    sk = jnp.sin(kr)
    ck = jnp.cos(kr)
    s0 = sk * inv_r
    s1 = kk * ck * inv_r - sk * (inv_r * inv_r)
    s2 = (-(kk * kk) * sk * inv_r - 2.0 * kk * ck * (inv_r * inv_r)
          + 2.0 * sk * (inv_r * inv_r * inv_r))
    mask = r < 1.0
    env0 = jnp.where(mask, (1.0 - r) ** 2, 0.0)
    env1 = jnp.where(mask, -2.0 * (1.0 - r), 0.0)
    env2 = jnp.where(mask, 2.0, 0.0)
    rb = s0 * env0
    fp = s1 * env0 + s0 * env1
    fpp = s2 * env0 + 2.0 * s1 * env1 + s0 * env2

    h = _dot(ohsp[...], wemb[...])
    us = []
    zs = []
    for l in range(3):
        u = (_dot_ex(smat[...], _dot(h, w1h[l]))
             + _dot(rb, w1rb[l]) + _dot(v, w1v[l]))
        us.append(u)
        a = u * jax.nn.sigmoid(u)
        agg = _dot_ex(rtmat[...], a) * _INV_AVG
        z = _dot(agg, w2[l]) + _dot(h, ws[l])
        zs.append(z)
        h = z * jax.nn.sigmoid(z)

    ones_n = jnp.ones((N, 1), dtype=jnp.float32)
    gh = _dot(_dot(ones_n, wout2t[...]), wout1t[...])
    g_rb = jnp.zeros((E, NB), dtype=jnp.float32)
    for l in (2, 1, 0):
        spz, spz2 = _silu_d(zs[l])
        spz_o[l] = spz
        bz_o[l] = gh * spz2
        gz = gh * spz
        g_agg = _dot(gz, w2t[l])
        gh = _dot(gz, wst[l])
        g_a = _dot_ex(rmat[...], g_agg) * _INV_AVG
        spu, spu2 = _silu_d(us[l])
        spu_o[l] = spu
        bu_o[l] = g_a * spu2
        gu = g_a * spu
        gh = gh + _dot(_dot_ex(stmat[...], gu), w1ht[l])
        g_rb = g_rb + _dot(gu, w1rbt[l])

    c = jnp.sum(g_rb * fp, axis=1, keepdims=True)
    fp_o[...] = fp
    vhat_o[...] = v * inv_r
    cor_o[...] = c * inv_r
    d2_o[...] = jnp.sum(g_rb * fpp, axis=1, keepdims=True)


def _tangent_kernel(smat, stmat, rmat, rtmat, rstmat,
                    w1h, w1rb, w1v, w2, ws, w1ht, w1rbt, w1vt, w2t, wst,
                    spu, bu, spz, bz, fp_r, vhat_r, cor_r, d2_r, tv_r,
                    out):
    tv = tv_r[...]
    vhat = vhat_r[...]
    fp = fp_r[...]
    tr = jnp.sum(tv * vhat[:, None, :], axis=2)
    trb2 = (tr[:, :, None] * fp[:, None, :]).reshape(E * BC, NB)
    tv2 = tv.reshape(E * BC, 3)

    th = jnp.zeros((N, BC, F), dtype=jnp.float32)
    tus = []
    tzs = []
    for l in range(3):
        tu2 = _dot(trb2, w1rb[l]) + _dot(tv2, w1v[l])
        if l > 0:
            thw = _dot(th.reshape(N * BC, F), w1h[l]).reshape(N, BC * F)
            tu2 = tu2 + _dot_ex(smat[...], thw).reshape(E * BC, F)
        tu = tu2.reshape(E, BC, F)
        tus.append(tu)
        ta = spu[l][:, None, :] * tu
        tagg = (_dot_ex(rtmat[...], ta.reshape(E, BC * F)) * _INV_AVG)
        tz2 = _dot(tagg.reshape(N * BC, F), w2[l])
        if l > 0:
            tz2 = tz2 + _dot(th.reshape(N * BC, F), ws[l])
        tz = tz2.reshape(N, BC, F)
        tzs.append(tz)
        th = spz[l][:, None, :] * tz

    tgh = jnp.zeros((N, BC, F), dtype=jnp.float32)
    tgrb = jnp.zeros((E, BC, NB), dtype=jnp.float32)
    tgvd = jnp.zeros((E, BC, 3), dtype=jnp.float32)
    for l in (2, 1, 0):
        tgz = tgh * spz[l][:, None, :] + bz[l][:, None, :] * tzs[l]
        tgz2 = tgz.reshape(N * BC, F)
        tgagg = _dot(tgz2, w2t[l]).reshape(N, BC * F)
        tga = (_dot_ex(rmat[...], tgagg) * _INV_AVG).reshape(E, BC, F)
        tgu = tga * spu[l][:, None, :] + bu[l][:, None, :] * tus[l]
        tgu2 = tgu.reshape(E * BC, F)
        if l > 0:
            tgs = _dot_ex(stmat[...], tgu.reshape(E, BC * F))
            tgh = (_dot(tgz2, wst[l]).reshape(N, BC, F)
                   + _dot(tgs.reshape(N * BC, F), w1ht[l]).reshape(N, BC, F))
        tgrb = tgrb + _dot(tgu2, w1rbt[l]).reshape(E, BC, NB)
        tgvd = tgvd + _dot(tgu2, w1vt[l]).reshape(E, BC, 3)

    d2 = d2_r[...]
    cor = cor_r[...]
    tc = jnp.sum(tgrb * fp[:, None, :], axis=2) + d2 * tr
    tgv = (tgvd + tc[:, :, None] * vhat[:, None, :]
           + cor[:, :, None] * (tv - vhat[:, None, :] * tr[:, :, None]))
    tgv = tgv * (1.0 / CUTOFF)
    out[...] = _dot_ex(rstmat[...], tgv.reshape(E, BC * 3)).reshape(N, BC, 3)


def kernel(positions, senders, receivers, species, Wemb,
           W1_0, Wself_0, W2_0, W1_1, Wself_1, W2_1, W1_2, Wself_2, W2_2,
           Wout1, Wout2):
    f32 = jnp.float32
    S = jax.nn.one_hot(senders, N, dtype=f32)
    R = jax.nn.one_hot(receivers, N, dtype=f32)
    RS = R - S
    D = RS * (1.0 / CUTOFF)
    TV = (D[:, :, None, None] * jnp.eye(3, dtype=f32)).reshape(E, B, 3)
    OH = jax.nn.one_hot(species, 9, dtype=f32)

    W1s = (W1_0, W1_1, W1_2)
    w1h = jnp.stack([w[:F] for w in W1s])
    w1rb = jnp.stack([w[F:F + NB] for w in W1s])
    w1v = jnp.stack([w[F + NB:] for w in W1s])
    w2 = jnp.stack((W2_0, W2_1, W2_2))
    ws = jnp.stack((Wself_0, Wself_1, Wself_2))
    w1ht = w1h.transpose(0, 2, 1)
    w1rbt = w1rb.transpose(0, 2, 1)
    w1vt = w1v.transpose(0, 2, 1)
    w2t = w2.transpose(0, 2, 1)
    wst = ws.transpose(0, 2, 1)

    spu, bu, spz, bz, fp, vhat, cor, d2 = pl.pallas_call(
        _primal_kernel,
        out_shape=(
            jax.ShapeDtypeStruct((3, E, F), f32),
            jax.ShapeDtypeStruct((3, E, F), f32),
            jax.ShapeDtypeStruct((3, N, F), f32),
            jax.ShapeDtypeStruct((3, N, F), f32),
            jax.ShapeDtypeStruct((E, NB), f32),
            jax.ShapeDtypeStruct((E, 3), f32),
            jax.ShapeDtypeStruct((E, 1), f32),
            jax.ShapeDtypeStruct((E, 1), f32),
        ),
    )(positions, OH, Wemb, RS, S, S.T, R, R.T,
      w1h, w1rb, w1v, w2, ws, w1ht, w1rbt, w2t, wst,
      Wout1.T, Wout2.T)

    full = lambda shp: pl.BlockSpec(shp, lambda g: (0,) * len(shp))
    ins = (S, S.T, R, R.T, RS.T,
           w1h, w1rb, w1v, w2, ws, w1ht, w1rbt, w1vt, w2t, wst,
           spu, bu, spz, bz, fp, vhat, cor, d2)
    hout = pl.pallas_call(
        _tangent_kernel,
        grid=(B // BC,),
        in_specs=[full(x.shape) for x in ins]
        + [pl.BlockSpec((E, BC, 3), lambda g: (0, g, 0))],
        out_specs=pl.BlockSpec((N, BC, 3), lambda g: (0, g, 0)),
        out_shape=jax.ShapeDtypeStruct((N, B, 3), f32),
    )(*ins, TV)

    return hout.transpose(1, 0, 2).reshape(N, 3, N, 3)
